# baseline (speedup 1.0000x reference)
import jax
import jax.numpy as jnp
import numpy as np
from jax import lax
from jax.experimental import pallas as pl
from jax.experimental.pallas import tpu as pltpu
from jax.experimental.pallas import tpu_sc as plsc

_NUM_LEVELS = 16
_LEVEL_DIM = 2
_BASE_RES = 16
_LOG2_HASH = 19
_DESIRED_RES = 2048
_PER_LEVEL_SCALE = float(np.exp2(np.log2(_DESIRED_RES / _BASE_RES) / (_NUM_LEVELS - 1)))
_MAX_PARAMS = 2 ** _LOG2_HASH
_PRIMES = (1, 2654435761, 805459861)
_MAX_LEVELS = 12
_FEAT_DIM = _NUM_LEVELS * _LEVEL_DIM
_N_POINTS = 262144


def _level_params():
    out = []
    offset = 0
    for l in range(_NUM_LEVELS):
        scale = _BASE_RES * (_PER_LEVEL_SCALE ** l) - 1.0
        res = int(np.ceil(scale)) + 1
        n = min(_MAX_PARAMS, res ** 3)
        n = int(np.ceil(n / 8) * 8)
        out.append((scale, res, offset, n, (res ** 3) > n))
        offset += n
    return out, offset


_LEVELS, _TOTAL_PARAMS = _level_params()

_NC, _NS = 2, 16
_NW = _NC * _NS
_PPW = _N_POINTS // _NW
_CH = 128
_NCHUNK = _PPW // _CH
_NL = _MAX_LEVELS
_NCORN = _NL * 8
_EL = _NCORN * 2 * _CH

_P1 = int(np.uint32(_PRIMES[1]).astype(np.int32))
_P2 = int(np.uint32(_PRIMES[2]).astype(np.int32))


def _body(xs, table, out, xv, idx_a, idx_b, w_a, w_b, rows_a, rows_b,
          out_v, sem_a, sem_b):
    wid = lax.axis_index("s") * _NC + lax.axis_index("c")
    base = wid * _PPW

    iota = lax.iota(jnp.int32, 16)
    zf = jnp.zeros((16,), jnp.float32)

    tailpat = ((iota >> 3) * 32) + (iota & 7) + 24

    def zero_tail(i, c):
        plsc.store_scatter(out_v, [tailpat + i * 64], zf)
        return c

    lax.fori_loop(0, _CH // 2, zero_tail, 0)

    def compute_idx(ci, idx_v, w_v):
        gbase = pl.multiple_of(base + ci * _CH, _CH)
        for d in range(3):
            pltpu.sync_copy(
                xs.at[pl.ds(d * _N_POINTS + gbase, _CH)],
                xv.at[pl.ds(d * _CH, _CH)],
            )

        def jbody(j, c):
            p0 = pl.multiple_of(j * 16, 16)
            xc = xv[pl.ds(p0, 16)]
            yc = xv[pl.ds(_CH + p0, 16)]
            zc = xv[pl.ds(2 * _CH + p0, 16)]
            for l in range(_NL):
                scale, res, off, n, use_hash = _LEVELS[l]
                fr = []
                pgs = []
                for xd in (xc, yc, zc):
                    pos = xd * scale + 0.5
                    ti = pos.astype(jnp.int32)
                    fr.append(pos - ti.astype(jnp.float32))
                    pgs.append(jnp.clip(ti, 0, res - 2))
                if use_hash:
                    ax = (pgs[0], pgs[0] + 1)
                    ay0 = pgs[1] * _P1
                    ay = (ay0, ay0 + _P1)
                    az0 = pgs[2] * _P2
                    az = (az0, az0 + _P2)
                    hmask = n - 1
                else:
                    ax = (pgs[0], pgs[0] + 1)
                    ay0 = pgs[1] * res
                    ay = (ay0, ay0 + res)
                    az0 = pgs[2] * (res * res)
                    az = (az0, az0 + res * res)
                wx = (1.0 - fr[0], fr[0])
                wy = (1.0 - fr[1], fr[1])
                wz = (1.0 - fr[2], fr[2])
                for cx in range(2):
                    for cy in range(2):
                        wxy = wx[cx] * wy[cy]
                        if use_hash:
                            axy = ax[cx] ^ ay[cy]
                        else:
                            axy = ax[cx] + ay[cy]
                        for cz in range(2):
                            corner = cx + 2 * cy + 4 * cz
                            row = (l * 8 + corner) * 2
                            if use_hash:
                                idx = ((axy ^ az[cz]) & hmask) + off
                            else:
                                idx = axy + az[cz] + off
                            e0 = idx + idx
                            idx_v[pl.ds(row * _CH + p0, 16)] = e0
                            idx_v[pl.ds((row + 1) * _CH + p0, 16)] = e0 + 1
                            w_v[pl.ds((l * 8 + corner) * _CH + p0, 16)] = (
                                wxy * wz[cz]
                            )
            return c

        lax.fori_loop(0, _CH // 16, jbody, 0)

    def fire(idx_v, rows_v, sem):
        pltpu.make_async_copy(table.at[idx_v], rows_v, sem).start()

    def wait(idx_v, rows_v, sem):
        pltpu.make_async_copy(table.at[idx_v], rows_v, sem).wait()

    def combine(ci, w_v, rows_v):
        gbase = pl.multiple_of(base + ci * _CH, _CH)

        def cbody(j, c):
            p0 = pl.multiple_of(j * 16, 16)
            obase = iota * 32 + j * 512
            for l in range(_NL):
                acc0 = zf
                acc1 = zf
                for corner in range(8):
                    row = (l * 8 + corner) * 2
                    w = w_v[pl.ds((l * 8 + corner) * _CH + p0, 16)]
                    f0 = rows_v[pl.ds(row * _CH + p0, 16)]
                    f1 = rows_v[pl.ds((row + 1) * _CH + p0, 16)]
                    acc0 = acc0 + w * f0
                    acc1 = acc1 + w * f1
                plsc.store_scatter(out_v, [obase + (2 * l)], acc0)
                plsc.store_scatter(out_v, [obase + (2 * l + 1)], acc1)
            return c

        lax.fori_loop(0, _CH // 16, cbody, 0)
        pltpu.sync_copy(out_v, out.at[pl.ds(gbase * _FEAT_DIM, _CH * _FEAT_DIM)])

    compute_idx(0, idx_a, w_a)
    fire(idx_a, rows_a, sem_a)

    def pipe(k, carry):
        c0 = k * 2
        compute_idx(c0 + 1, idx_b, w_b)
        fire(idx_b, rows_b, sem_b)
        wait(idx_a, rows_a, sem_a)
        combine(c0, w_a, rows_a)

        @pl.when(k < (_NCHUNK // 2 - 1))
        def _():
            compute_idx(c0 + 2, idx_a, w_a)
            fire(idx_a, rows_a, sem_a)

        wait(idx_b, rows_b, sem_b)
        combine(c0 + 1, w_b, rows_b)
        return carry

    lax.fori_loop(0, _NCHUNK // 2, pipe, 0)


@jax.jit
def _hash_grid_sc(xs, table):
    mesh = plsc.VectorSubcoreMesh(
        core_axis_name="c", subcore_axis_name="s", num_cores=_NC, num_subcores=_NS
    )
    return pl.kernel(
        _body,
        out_type=jax.ShapeDtypeStruct((_N_POINTS * _FEAT_DIM,), jnp.float32),
        mesh=mesh,
        compiler_params=pltpu.CompilerParams(
            needs_layout_passes=False, use_tc_tiling_on_sc=False
        ),
        scratch_types=[
            pltpu.VMEM((3 * _CH,), jnp.float32),
            pltpu.VMEM((_EL,), jnp.int32),
            pltpu.VMEM((_EL,), jnp.int32),
            pltpu.VMEM((_EL // 2,), jnp.float32),
            pltpu.VMEM((_EL // 2,), jnp.float32),
            pltpu.VMEM((_EL,), jnp.float32),
            pltpu.VMEM((_EL,), jnp.float32),
            pltpu.VMEM((_CH * _FEAT_DIM,), jnp.float32),
            pltpu.SemaphoreType.DMA,
            pltpu.SemaphoreType.DMA,
        ],
        name="hash_grid_sc",
    )(xs, table)


def kernel(x, step, table):
    level = jnp.minimum(step // 1000 + 1, _MAX_LEVELS)
    mask = (jnp.arange(_FEAT_DIM) < level * 2).astype(jnp.float32)[None, :]
    xs = x.T.reshape(-1)
    out = _hash_grid_sc(xs, table.reshape(-1))
    return out.reshape(_N_POINTS, _FEAT_DIM) * mask


# speedup vs baseline: 3.1245x; 3.1245x over previous
import jax
import jax.numpy as jnp
import numpy as np
from jax import lax
from jax.experimental import pallas as pl
from jax.experimental.pallas import tpu as pltpu
from jax.experimental.pallas import tpu_sc as plsc

_NUM_LEVELS = 16
_LEVEL_DIM = 2
_BASE_RES = 16
_LOG2_HASH = 19
_DESIRED_RES = 2048
_PER_LEVEL_SCALE = float(np.exp2(np.log2(_DESIRED_RES / _BASE_RES) / (_NUM_LEVELS - 1)))
_MAX_PARAMS = 2 ** _LOG2_HASH
_PRIMES = (1, 2654435761, 805459861)
_MAX_LEVELS = 12
_FEAT_DIM = _NUM_LEVELS * _LEVEL_DIM
_N_POINTS = 262144


def _level_params():
    out = []
    offset = 0
    for l in range(_NUM_LEVELS):
        scale = _BASE_RES * (_PER_LEVEL_SCALE ** l) - 1.0
        res = int(np.ceil(scale)) + 1
        n = min(_MAX_PARAMS, res ** 3)
        n = int(np.ceil(n / 8) * 8)
        out.append((scale, res, offset, n, (res ** 3) > n))
        offset += n
    return out, offset


_LEVELS, _TOTAL_PARAMS = _level_params()

_NC, _NS = 2, 16
_NW = _NC * _NS
_PPW = _N_POINTS // _NW
_CH = 128
_NCHUNK = _PPW // _CH
_NL = _MAX_LEVELS
_NCORN = _NL * 8
_EL = _NCORN * 2 * _CH

_P1 = int(np.uint32(_PRIMES[1]).astype(np.int32))
_P2 = int(np.uint32(_PRIMES[2]).astype(np.int32))


def _body(xs, table, out, xv, idx_a, idx_b, w_a, w_b, rows_a, rows_b,
          out_v, sem_a, sem_b):
    wid = lax.axis_index("s") * _NC + lax.axis_index("c")
    base = wid * _PPW

    iota = lax.iota(jnp.int32, 16)
    zf = jnp.zeros((16,), jnp.float32)

    tailpat = ((iota >> 3) * 32) + (iota & 7) + 24

    def zero_tail(i, c):
        plsc.store_scatter(out_v, [tailpat + i * 64], zf)
        return c

    lax.fori_loop(0, _CH // 2, zero_tail, 0)

    def compute_idx(ci, idx_v, w_v):
        gbase = pl.multiple_of(base + ci * _CH, _CH)
        for d in range(3):
            pltpu.sync_copy(
                xs.at[pl.ds(d * _N_POINTS + gbase, _CH)],
                xv.at[pl.ds(d * _CH, _CH)],
            )

        def jbody(j, c):
            p0 = pl.multiple_of(j * 16, 16)
            xc = xv[pl.ds(p0, 16)]
            yc = xv[pl.ds(_CH + p0, 16)]
            zc = xv[pl.ds(2 * _CH + p0, 16)]
            for l in range(_NL):
                scale, res, off, n, use_hash = _LEVELS[l]
                fr = []
                pgs = []
                for xd in (xc, yc, zc):
                    pos = xd * scale + 0.5
                    ti = pos.astype(jnp.int32)
                    fr.append(pos - ti.astype(jnp.float32))
                    pgs.append(jnp.clip(ti, 0, res - 2))
                if use_hash:
                    ax = (pgs[0], pgs[0] + 1)
                    ay0 = pgs[1] * _P1
                    ay = (ay0, ay0 + _P1)
                    az0 = pgs[2] * _P2
                    az = (az0, az0 + _P2)
                    hmask = n - 1
                else:
                    ax = (pgs[0], pgs[0] + 1)
                    ay0 = pgs[1] * res
                    ay = (ay0, ay0 + res)
                    az0 = pgs[2] * (res * res)
                    az = (az0, az0 + res * res)
                wx = (1.0 - fr[0], fr[0])
                wy = (1.0 - fr[1], fr[1])
                wz = (1.0 - fr[2], fr[2])
                for cx in range(2):
                    for cy in range(2):
                        wxy = wx[cx] * wy[cy]
                        if use_hash:
                            axy = ax[cx] ^ ay[cy]
                        else:
                            axy = ax[cx] + ay[cy]
                        for cz in range(2):
                            corner = cx + 2 * cy + 4 * cz
                            row = (l * 8 + corner) * 2
                            if use_hash:
                                idx = ((axy ^ az[cz]) & hmask) + off
                            else:
                                idx = axy + az[cz] + off
                            idx_v[pl.ds(row * _CH + p0, 16)] = idx
                            idx_v[pl.ds((row + 1) * _CH + p0, 16)] = (
                                idx + _TOTAL_PARAMS
                            )
                            w_v[pl.ds((l * 8 + corner) * _CH + p0, 16)] = (
                                wxy * wz[cz]
                            )
            return c

        lax.fori_loop(0, _CH // 16, jbody, 0)

    def fire(idx_v, rows_v, sem):
        pltpu.make_async_copy(table.at[idx_v], rows_v, sem).start()

    def wait(idx_v, rows_v, sem):
        pltpu.make_async_copy(table.at[idx_v], rows_v, sem).wait()

    def combine(ci, w_v, rows_v):
        gbase = pl.multiple_of(base + ci * _CH, _CH)

        def cbody(j, c):
            p0 = pl.multiple_of(j * 16, 16)
            obase = iota * 32 + j * 512
            for l in range(_NL):
                acc0 = zf
                acc1 = zf
                for corner in range(8):
                    row = (l * 8 + corner) * 2
                    w = w_v[pl.ds((l * 8 + corner) * _CH + p0, 16)]
                    f0 = rows_v[pl.ds(row * _CH + p0, 16)]
                    f1 = rows_v[pl.ds((row + 1) * _CH + p0, 16)]
                    acc0 = acc0 + w * f0
                    acc1 = acc1 + w * f1
                plsc.store_scatter(out_v, [obase + (2 * l)], acc0)
                plsc.store_scatter(out_v, [obase + (2 * l + 1)], acc1)
            return c

        lax.fori_loop(0, _CH // 16, cbody, 0)
        pltpu.sync_copy(out_v, out.at[pl.ds(gbase * _FEAT_DIM, _CH * _FEAT_DIM)])

    compute_idx(0, idx_a, w_a)
    fire(idx_a, rows_a, sem_a)

    def pipe(k, carry):
        c0 = k * 2
        compute_idx(c0 + 1, idx_b, w_b)
        fire(idx_b, rows_b, sem_b)
        wait(idx_a, rows_a, sem_a)
        combine(c0, w_a, rows_a)

        @pl.when(k < (_NCHUNK // 2 - 1))
        def _():
            compute_idx(c0 + 2, idx_a, w_a)
            fire(idx_a, rows_a, sem_a)

        wait(idx_b, rows_b, sem_b)
        combine(c0 + 1, w_b, rows_b)
        return carry

    lax.fori_loop(0, _NCHUNK // 2, pipe, 0)


@jax.jit
def _hash_grid_sc(xs, table):
    mesh = plsc.VectorSubcoreMesh(
        core_axis_name="c", subcore_axis_name="s", num_cores=_NC, num_subcores=_NS
    )
    return pl.kernel(
        _body,
        out_type=jax.ShapeDtypeStruct((_N_POINTS * _FEAT_DIM,), jnp.float32),
        mesh=mesh,
        compiler_params=pltpu.CompilerParams(
            needs_layout_passes=False, use_tc_tiling_on_sc=False
        ),
        scratch_types=[
            pltpu.VMEM((3 * _CH,), jnp.float32),
            pltpu.VMEM((_EL,), jnp.int32),
            pltpu.VMEM((_EL,), jnp.int32),
            pltpu.VMEM((_EL // 2,), jnp.float32),
            pltpu.VMEM((_EL // 2,), jnp.float32),
            pltpu.VMEM((_EL,), jnp.float32),
            pltpu.VMEM((_EL,), jnp.float32),
            pltpu.VMEM((_CH * _FEAT_DIM,), jnp.float32),
            pltpu.SemaphoreType.DMA,
            pltpu.SemaphoreType.DMA,
        ],
        name="hash_grid_sc",
    )(xs, table)


def kernel(x, step, table):
    level = jnp.minimum(step // 1000 + 1, _MAX_LEVELS)
    mask = (jnp.arange(_FEAT_DIM) < level * 2).astype(jnp.float32)[None, :]
    xs = x.T.reshape(-1)
    out = _hash_grid_sc(xs, table.T.reshape(-1))
    return out.reshape(_N_POINTS, _FEAT_DIM) * mask
